# baseline (speedup 1.0000x reference)
import jax
import jax.numpy as jnp
from jax import lax
from jax.experimental import pallas as pl
from jax.experimental.pallas import tpu as pltpu
from jax.experimental.pallas import tpu_sc as plsc

VOCAB = 100000
EMBED_DIM = 64
BATCH = 4096
SEQ = 200

NC = 2
NS = 16
NW = NC * NS
BBLK = BATCH // NW
L = 16


def _transpose_block(src, dst):
    lanes = jnp.arange(L, dtype=jnp.int32)

    def body(te, _):
        for er in range(8):
            e = te * 8 + er
            col = jnp.full((L,), 0, dtype=jnp.int32) + e
            for k in range(BBLK // L):
                rows = lanes + (L * k)
                v = plsc.load_gather(src, [rows, col])
                dst[te, er, pl.ds(L * k, L)] = v
        return 0

    lax.fori_loop(0, 8, body, 0, unroll=False)


def _emb_kernel(idx_hbm, table_hbm, out_hbm, idx_raw, idx_t, rows_v, tile_v,
                gsems, osems):
    wid = lax.axis_index("s") * NC + lax.axis_index("c")

    pltpu.sync_copy(idx_hbm.at[wid], idx_raw)
    lanes = jnp.arange(L, dtype=jnp.int32)

    def tbody(s, _):
        scol = jnp.full((L,), 0, dtype=jnp.int32) + s
        for k in range(BBLK // L):
            v = plsc.load_gather(idx_raw, [lanes + (L * k), scol])
            idx_t[s, pl.ds(L * k, L)] = v
        return 0

    lax.fori_loop(0, SEQ, tbody, 0, unroll=False)

    def fire_gather(s, b):
        pltpu.async_copy(table_hbm.at[idx_t.at[s]], rows_v.at[b], gsems.at[b])

    def wait_gather(b):
        pltpu.make_async_copy(table_hbm.at[idx_t.at[0]], rows_v.at[b],
                              gsems.at[b]).wait()

    def fire_out(s, b):
        pltpu.async_copy(tile_v.at[b], out_hbm.at[s, :, wid], osems.at[b])

    def wait_out(b):
        pltpu.make_async_copy(tile_v.at[b], out_hbm.at[0, :, wid],
                              osems.at[b]).wait()

    fire_gather(0, 0)
    fire_gather(1, 1)

    def body(i, _):
        for b in range(2):
            s = i * 2 + b
            wait_gather(b)

            @pl.when(i > 0)
            def _():
                wait_out(b)

            _transpose_block(rows_v.at[b], tile_v.at[b])
            fire_out(s, b)

            @pl.when(i < SEQ // 2 - 1)
            def _():
                fire_gather(s + 2, b)

        return 0

    lax.fori_loop(0, SEQ // 2, body, 0, unroll=False)
    wait_out(0)
    wait_out(1)


@jax.jit
def kernel(tokenized_sentence, table):
    idx = tokenized_sentence.astype(jnp.int32).reshape(NW, BBLK, SEQ)
    mesh = plsc.VectorSubcoreMesh(core_axis_name="c", subcore_axis_name="s")
    out = pl.kernel(
        _emb_kernel,
        out_type=jax.ShapeDtypeStruct((SEQ, 8, NW, 8, 128), jnp.float32),
        mesh=mesh,
        scratch_types=[
            pltpu.VMEM((BBLK, SEQ), jnp.int32),
            pltpu.VMEM((SEQ, BBLK), jnp.int32),
            pltpu.VMEM((2, BBLK, EMBED_DIM), jnp.float32),
            pltpu.VMEM((2, 8, 8, 128), jnp.float32),
            pltpu.SemaphoreType.DMA((2,)),
            pltpu.SemaphoreType.DMA((2,)),
        ],
        compiler_params=pltpu.CompilerParams(use_tc_tiling_on_sc=False, needs_layout_passes=False),
    )(idx, table)
    return out.transpose(2, 4, 0, 1, 3).reshape(BATCH, SEQ, EMBED_DIM)


# speedup vs baseline: 2.6159x; 2.6159x over previous
import jax
import jax.numpy as jnp
from jax import lax
from jax.experimental import pallas as pl
from jax.experimental.pallas import tpu as pltpu
from jax.experimental.pallas import tpu_sc as plsc

VOCAB = 100000
EMBED_DIM = 64
BATCH = 4096
SEQ = 200

NC = 2
NS = 16
NW = NC * NS
BBLK = BATCH // NW
L = 16


def _transpose_block(src, dst):
    lanes = jnp.arange(L, dtype=jnp.int32)

    def sub(i, _):
        bc0 = (i >> 2) * L
        e0 = (i & 3) * L
        row_v = lanes + bc0
        for j in range(L):
            col_v = ((lanes + j) & (L - 1)) + e0
            v = plsc.load_gather(src, [row_v, col_v])
            plsc.store_scatter(dst, [col_v >> 3, col_v & 7, row_v], v)
        return 0

    lax.fori_loop(0, 32, sub, 0, unroll=False)


def _emb_kernel(idx_hbm, table_hbm, out_hbm, idx_raw, idx_t, rows_v, tile_v,
                gsems, osems):
    wid = lax.axis_index("s") * NC + lax.axis_index("c")

    pltpu.sync_copy(idx_hbm.at[wid], idx_raw)
    lanes = jnp.arange(L, dtype=jnp.int32)

    def tbody(s, _):
        scol = jnp.full((L,), 0, dtype=jnp.int32) + s
        for k in range(BBLK // L):
            v = plsc.load_gather(idx_raw, [lanes + (L * k), scol])
            idx_t[s, pl.ds(L * k, L)] = v
        return 0

    lax.fori_loop(0, SEQ, tbody, 0, unroll=False)

    def fire_gather(s, b):
        pltpu.async_copy(table_hbm.at[idx_t.at[s]], rows_v.at[b], gsems.at[b])

    def wait_gather(b):
        pltpu.make_async_copy(table_hbm.at[idx_t.at[0]], rows_v.at[b],
                              gsems.at[b]).wait()

    def fire_out(s, b):
        pltpu.async_copy(tile_v.at[b], out_hbm.at[s, :, wid], osems.at[b])

    def wait_out(b):
        pltpu.make_async_copy(tile_v.at[b], out_hbm.at[0, :, wid],
                              osems.at[b]).wait()

    fire_gather(0, 0)
    fire_gather(1, 1)

    def body(i, _):
        for b in range(2):
            s = i * 2 + b
            wait_gather(b)

            @pl.when(i > 0)
            def _():
                wait_out(b)

            _transpose_block(rows_v.at[b], tile_v.at[b])
            fire_out(s, b)

            @pl.when(i < SEQ // 2 - 1)
            def _():
                fire_gather(s + 2, b)

        return 0

    lax.fori_loop(0, SEQ // 2, body, 0, unroll=False)
    wait_out(0)
    wait_out(1)


@jax.jit
def kernel(tokenized_sentence, table):
    idx = tokenized_sentence.astype(jnp.int32).reshape(NW, BBLK, SEQ)
    mesh = plsc.VectorSubcoreMesh(core_axis_name="c", subcore_axis_name="s")
    out = pl.kernel(
        _emb_kernel,
        out_type=jax.ShapeDtypeStruct((SEQ, 8, NW, 8, 128), jnp.float32),
        mesh=mesh,
        scratch_types=[
            pltpu.VMEM((BBLK, SEQ), jnp.int32),
            pltpu.VMEM((SEQ, BBLK), jnp.int32),
            pltpu.VMEM((2, BBLK, EMBED_DIM), jnp.float32),
            pltpu.VMEM((2, 8, 8, 128), jnp.float32),
            pltpu.SemaphoreType.DMA((2,)),
            pltpu.SemaphoreType.DMA((2,)),
        ],
        compiler_params=pltpu.CompilerParams(use_tc_tiling_on_sc=False, needs_layout_passes=False),
    )(idx, table)
    return out.transpose(2, 4, 0, 1, 3).reshape(BATCH, SEQ, EMBED_DIM)


# speedup vs baseline: 4.2453x; 1.6229x over previous
import jax
import jax.numpy as jnp
from jax import lax
from jax.experimental import pallas as pl
from jax.experimental.pallas import tpu as pltpu
from jax.experimental.pallas import tpu_sc as plsc

VOCAB = 100000
EMBED_DIM = 64
BATCH = 4096
SEQ = 200

NC = 2
NS = 16
NW = NC * NS
BBLK = BATCH // NW
L = 16


def _transpose_block(src, dst):
    lanes = jnp.arange(L, dtype=jnp.int32)

    @plsc.parallel_loop(0, 32)
    def sub(i):
        bc0 = (i >> 2) * L
        e0 = (i & 3) * L
        row_v = lanes + bc0
        for j in range(L):
            col_v = ((lanes + j) & (L - 1)) + e0
            v = plsc.load_gather(src, [row_v, col_v])
            plsc.store_scatter(dst, [col_v >> 3, col_v & 7, row_v], v)


def _emb_kernel(idx_hbm, table_hbm, out_hbm, idx_raw, idx_t, rows_v, tile_v,
                gsems, osems):
    wid = lax.axis_index("s") * NC + lax.axis_index("c")

    pltpu.sync_copy(idx_hbm.at[wid], idx_raw)
    lanes = jnp.arange(L, dtype=jnp.int32)

    @plsc.parallel_loop(0, SEQ)
    def tbody(s):
        scol = jnp.full((L,), 0, dtype=jnp.int32) + s
        for k in range(BBLK // L):
            v = plsc.load_gather(idx_raw, [lanes + (L * k), scol])
            idx_t[s, pl.ds(L * k, L)] = v

    def fire_gather(s, b):
        pltpu.async_copy(table_hbm.at[idx_t.at[s]], rows_v.at[b], gsems.at[b])

    def wait_gather(b):
        pltpu.make_async_copy(table_hbm.at[idx_t.at[0]], rows_v.at[b],
                              gsems.at[b]).wait()

    def fire_out(s, b):
        pltpu.async_copy(tile_v.at[b], out_hbm.at[s, :, wid], osems.at[b])

    def wait_out(b):
        pltpu.make_async_copy(tile_v.at[b], out_hbm.at[0, :, wid],
                              osems.at[b]).wait()

    fire_gather(0, 0)
    fire_gather(1, 1)

    def body(i, _):
        for b in range(2):
            s = i * 2 + b
            wait_gather(b)

            @pl.when(i > 0)
            def _():
                wait_out(b)

            _transpose_block(rows_v.at[b], tile_v.at[b])
            fire_out(s, b)

            @pl.when(i < SEQ // 2 - 1)
            def _():
                fire_gather(s + 2, b)

        return 0

    lax.fori_loop(0, SEQ // 2, body, 0, unroll=False)
    wait_out(0)
    wait_out(1)


@jax.jit
def kernel(tokenized_sentence, table):
    idx = tokenized_sentence.astype(jnp.int32).reshape(NW, BBLK, SEQ)
    mesh = plsc.VectorSubcoreMesh(core_axis_name="c", subcore_axis_name="s")
    out = pl.kernel(
        _emb_kernel,
        out_type=jax.ShapeDtypeStruct((SEQ, 8, NW, 8, 128), jnp.float32),
        mesh=mesh,
        scratch_types=[
            pltpu.VMEM((BBLK, SEQ), jnp.int32),
            pltpu.VMEM((SEQ, BBLK), jnp.int32),
            pltpu.VMEM((2, BBLK, EMBED_DIM), jnp.float32),
            pltpu.VMEM((2, 8, 8, 128), jnp.float32),
            pltpu.SemaphoreType.DMA((2,)),
            pltpu.SemaphoreType.DMA((2,)),
        ],
        compiler_params=pltpu.CompilerParams(use_tc_tiling_on_sc=False, needs_layout_passes=False),
    )(idx, table)
    return out.transpose(2, 4, 0, 1, 3).reshape(BATCH, SEQ, EMBED_DIM)


# speedup vs baseline: 4.4033x; 1.0372x over previous
import jax
import jax.numpy as jnp
from jax import lax
from jax.experimental import pallas as pl
from jax.experimental.pallas import tpu as pltpu
from jax.experimental.pallas import tpu_sc as plsc

VOCAB = 100000
EMBED_DIM = 64
BATCH = 4096
SEQ = 200

NC = 2
NS = 16
NW = NC * NS
BBLK = BATCH // NW
L = 16


def _transpose_block(src, dst):
    lanes = jnp.arange(L, dtype=jnp.int32)
    @plsc.parallel_loop(0, 32, unroll=2)
    def sub(i):
        bc0 = (i >> 2) * L
        e0 = (i & 3) * L
        row_v = lanes + bc0
        for j in range(L):
            col_v = ((lanes + j) & (L - 1)) + e0
            v = plsc.load_gather(src, [row_v, col_v])
            plsc.store_scatter(dst, [col_v >> 3, col_v & 7, row_v], v)


def _emb_kernel(idx_hbm, table_hbm, out_hbm, idx_raw, idx_t, rows_v, tile_v,
                gsems, osems):
    wid = lax.axis_index("s") * NC + lax.axis_index("c")

    pltpu.sync_copy(idx_hbm.at[wid], idx_raw)
    lanes = jnp.arange(L, dtype=jnp.int32)

    @plsc.parallel_loop(0, SEQ)
    def tbody(s):
        scol = jnp.full((L,), 0, dtype=jnp.int32) + s
        for k in range(BBLK // L):
            v = plsc.load_gather(idx_raw, [lanes + (L * k), scol])
            idx_t[s, pl.ds(L * k, L)] = v

    def fire_gather(s, b):
        pltpu.async_copy(table_hbm.at[idx_t.at[s]], rows_v.at[b], gsems.at[b])

    def wait_gather(b):
        pltpu.make_async_copy(table_hbm.at[idx_t.at[0]], rows_v.at[b],
                              gsems.at[b]).wait()

    def fire_out(s, b):
        pltpu.async_copy(tile_v.at[b], out_hbm.at[s, :, wid], osems.at[b])

    def wait_out(b):
        pltpu.make_async_copy(tile_v.at[b], out_hbm.at[0, :, wid],
                              osems.at[b]).wait()

    fire_gather(0, 0)
    fire_gather(1, 1)

    def body(i, _):
        for b in range(2):
            s = i * 2 + b
            wait_gather(b)

            @pl.when(i > 0)
            def _():
                wait_out(b)

            _transpose_block(rows_v.at[b], tile_v.at[b])
            fire_out(s, b)

            @pl.when(i < SEQ // 2 - 1)
            def _():
                fire_gather(s + 2, b)

        return 0

    lax.fori_loop(0, SEQ // 2, body, 0, unroll=False)
    wait_out(0)
    wait_out(1)


@jax.jit
def kernel(tokenized_sentence, table):
    idx = tokenized_sentence.astype(jnp.int32).reshape(NW, BBLK, SEQ)
    mesh = plsc.VectorSubcoreMesh(core_axis_name="c", subcore_axis_name="s")
    out = pl.kernel(
        _emb_kernel,
        out_type=jax.ShapeDtypeStruct((SEQ, 8, NW, 8, 128), jnp.float32),
        mesh=mesh,
        scratch_types=[
            pltpu.VMEM((BBLK, SEQ), jnp.int32),
            pltpu.VMEM((SEQ, BBLK), jnp.int32),
            pltpu.VMEM((2, BBLK, EMBED_DIM), jnp.float32),
            pltpu.VMEM((2, 8, 8, 128), jnp.float32),
            pltpu.SemaphoreType.DMA((2,)),
            pltpu.SemaphoreType.DMA((2,)),
        ],
        compiler_params=pltpu.CompilerParams(use_tc_tiling_on_sc=False, needs_layout_passes=False),
    )(idx, table)
    return out.transpose(2, 4, 0, 1, 3).reshape(BATCH, SEQ, EMBED_DIM)


# speedup vs baseline: 4.7670x; 1.0826x over previous
import jax
import jax.numpy as jnp
from jax import lax
from jax.experimental import pallas as pl
from jax.experimental.pallas import tpu as pltpu
from jax.experimental.pallas import tpu_sc as plsc

VOCAB = 100000
EMBED_DIM = 64
BATCH = 4096
SEQ = 200

NC = 2
NS = 16
NW = NC * NS
BBLK = BATCH // NW
L = 16


def _transpose_block(src, dst):
    lanes = jnp.arange(L, dtype=jnp.int32)
    @plsc.parallel_loop(0, 32, unroll=2)
    def sub(i):
        bc0 = (i >> 2) * L
        e0 = (i & 3) * L
        row_v = lanes + bc0
        for j in range(L):
            col_v = ((lanes + j) & (L - 1)) + e0
            v = plsc.load_gather(src, [row_v, col_v])
            plsc.store_scatter(dst, [col_v, row_v], v)


def _emb_kernel(idx_hbm, table_hbm, out_hbm, idx_raw, idx_t, rows_v, tile_v,
                gsems, osems):
    wid = lax.axis_index("s") * NC + lax.axis_index("c")

    pltpu.sync_copy(idx_hbm.at[wid], idx_raw)
    lanes = jnp.arange(L, dtype=jnp.int32)

    @plsc.parallel_loop(0, SEQ)
    def tbody(s):
        scol = jnp.full((L,), 0, dtype=jnp.int32) + s
        for k in range(BBLK // L):
            v = plsc.load_gather(idx_raw, [lanes + (L * k), scol])
            idx_t[s, pl.ds(L * k, L)] = v

    def fire_gather(s, b):
        pltpu.async_copy(table_hbm.at[idx_t.at[s]], rows_v.at[b], gsems.at[b])

    def wait_gather(b):
        pltpu.make_async_copy(table_hbm.at[idx_t.at[0]], rows_v.at[b],
                              gsems.at[b]).wait()

    def fire_out(s, b):
        for te in range(8):
            pltpu.async_copy(tile_v.at[b, pl.ds(te * 8, 8)],
                             out_hbm.at[s, te, wid], osems.at[b])

    def wait_out(b):
        for te in range(8):
            pltpu.make_async_copy(tile_v.at[b, pl.ds(0, 8)],
                                  out_hbm.at[0, 0, wid], osems.at[b]).wait()

    fire_gather(0, 0)
    fire_gather(1, 1)

    def body(i, _):
        for b in range(2):
            s = i * 2 + b
            wait_gather(b)

            @pl.when(i > 0)
            def _():
                wait_out(b)

            _transpose_block(rows_v.at[b], tile_v.at[b])
            fire_out(s, b)

            @pl.when(i < SEQ // 2 - 1)
            def _():
                fire_gather(s + 2, b)

        return 0

    lax.fori_loop(0, SEQ // 2, body, 0, unroll=False)
    wait_out(0)
    wait_out(1)


@jax.jit
def kernel(tokenized_sentence, table):
    idx = tokenized_sentence.astype(jnp.int32).reshape(NW, BBLK, SEQ)
    mesh = plsc.VectorSubcoreMesh(core_axis_name="c", subcore_axis_name="s")
    out = pl.kernel(
        _emb_kernel,
        out_type=jax.ShapeDtypeStruct((SEQ, 8, NW, 8, 128), jnp.float32),
        mesh=mesh,
        scratch_types=[
            pltpu.VMEM((BBLK, SEQ), jnp.int32),
            pltpu.VMEM((SEQ, BBLK), jnp.int32),
            pltpu.VMEM((2, BBLK, EMBED_DIM), jnp.float32),
            pltpu.VMEM((2, EMBED_DIM, BBLK), jnp.float32),
            pltpu.SemaphoreType.DMA((2,)),
            pltpu.SemaphoreType.DMA((2,)),
        ],
        compiler_params=pltpu.CompilerParams(use_tc_tiling_on_sc=False, needs_layout_passes=False),
    )(idx, table)
    return out.transpose(2, 4, 0, 1, 3).reshape(BATCH, SEQ, EMBED_DIM)


# speedup vs baseline: 5.1337x; 1.0769x over previous
import jax
import jax.numpy as jnp
from jax import lax
from jax.experimental import pallas as pl
from jax.experimental.pallas import tpu as pltpu
from jax.experimental.pallas import tpu_sc as plsc

VOCAB = 100000
EMBED_DIM = 64
BATCH = 4096
SEQ = 200

NC = 2
NS = 16
NW = NC * NS
BBLK = BATCH // NW
L = 16


def _transpose_block(src, dst):
    lanes = jnp.arange(L, dtype=jnp.int32)
    @plsc.parallel_loop(0, 32, unroll=2)
    def sub(i):
        bc0 = (i >> 2) * L
        e0 = (i & 3) * L
        row_v = lanes + bc0
        dbase = e0 * BBLK + bc0
        for j in range(L):
            diag = (lanes + j) & (L - 1)
            v = plsc.load_gather(src, [row_v, diag + e0])
            plsc.store_scatter(dst, [diag * BBLK + lanes + dbase], v)


def _emb_kernel(idx_hbm, table_hbm, out_hbm, idx_raw, idx_t, rows_v, tile_v,
                gsems, osems):
    wid = lax.axis_index("s") * NC + lax.axis_index("c")

    pltpu.sync_copy(idx_hbm.at[wid], idx_raw)
    lanes = jnp.arange(L, dtype=jnp.int32)

    @plsc.parallel_loop(0, SEQ)
    def tbody(s):
        scol = jnp.full((L,), 0, dtype=jnp.int32) + s
        for k in range(BBLK // L):
            v = plsc.load_gather(idx_raw, [lanes + (L * k), scol])
            idx_t[s, pl.ds(L * k, L)] = v

    def fire_gather(s, b):
        pltpu.async_copy(table_hbm.at[idx_t.at[s]], rows_v.at[b], gsems.at[b])

    def wait_gather(b):
        pltpu.make_async_copy(table_hbm.at[idx_t.at[0]], rows_v.at[b],
                              gsems.at[b]).wait()

    def fire_out(s, b):
        for te in range(8):
            pltpu.async_copy(tile_v.at[b, pl.ds(te * 1024, 1024)],
                             out_hbm.at[s, te, wid], osems.at[b])

    def wait_out(b):
        for te in range(8):
            pltpu.make_async_copy(tile_v.at[b, pl.ds(0, 1024)],
                                  out_hbm.at[0, 0, wid], osems.at[b]).wait()

    fire_gather(0, 0)
    fire_gather(1, 1)

    def body(i, _):
        for b in range(2):
            s = i * 2 + b
            wait_gather(b)

            @pl.when(i > 0)
            def _():
                wait_out(b)

            _transpose_block(rows_v.at[b], tile_v.at[b])
            fire_out(s, b)

            @pl.when(i < SEQ // 2 - 1)
            def _():
                fire_gather(s + 2, b)

        return 0

    lax.fori_loop(0, SEQ // 2, body, 0, unroll=False)
    wait_out(0)
    wait_out(1)


@jax.jit
def kernel(tokenized_sentence, table):
    idx = tokenized_sentence.astype(jnp.int32).reshape(NW, BBLK, SEQ)
    mesh = plsc.VectorSubcoreMesh(core_axis_name="c", subcore_axis_name="s")
    out = pl.kernel(
        _emb_kernel,
        out_type=jax.ShapeDtypeStruct((SEQ, 8, NW, 1024), jnp.float32),
        mesh=mesh,
        scratch_types=[
            pltpu.VMEM((BBLK, SEQ), jnp.int32),
            pltpu.VMEM((SEQ, BBLK), jnp.int32),
            pltpu.VMEM((2, BBLK, EMBED_DIM), jnp.float32),
            pltpu.VMEM((2, EMBED_DIM * BBLK), jnp.float32),
            pltpu.SemaphoreType.DMA((2,)),
            pltpu.SemaphoreType.DMA((2,)),
        ],
        compiler_params=pltpu.CompilerParams(use_tc_tiling_on_sc=False, needs_layout_passes=False),
    )(idx, table)
    out5 = out.reshape(SEQ, 8, NW, 8, 128)
    return out5.transpose(2, 4, 0, 1, 3).reshape(BATCH, SEQ, EMBED_DIM)
